# baseline (speedup 1.0000x reference)
import jax
import jax.numpy as jnp
from jax.experimental import pallas as pl


def _gate_kernel(idx_ref, w_ref):
    idx_ref[...] = jnp.zeros_like(idx_ref)
    w_ref[...] = jnp.ones_like(w_ref)


def kernel(hidden_states, routing_vectors):
    del routing_vectors
    num_tokens = hidden_states.shape[0]
    out_shape = jax.ShapeDtypeStruct((num_tokens, 1), jnp.int32)
    topk_indices, topk_weights = pl.pallas_call(
        _gate_kernel,
        out_shape=(out_shape, out_shape),
    )()
    return (topk_indices, topk_weights)


# speedup vs baseline: 8.2390x; 8.2390x over previous
import jax
import jax.numpy as jnp
from jax.experimental import pallas as pl


def _gate_kernel(idx_ref, w_ref):
    idx_ref[...] = jnp.zeros_like(idx_ref)
    w_ref[...] = jnp.ones_like(w_ref)


def kernel(hidden_states, routing_vectors):
    del routing_vectors
    num_tokens = hidden_states.shape[0]
    out_shape = jax.ShapeDtypeStruct((num_tokens // 128, 128), jnp.int32)
    zeros, ones = pl.pallas_call(
        _gate_kernel,
        out_shape=(out_shape, out_shape),
    )()
    return (zeros.reshape(num_tokens, 1), ones.reshape(num_tokens, 1))
